# baseline (speedup 1.0000x reference)
import functools
import numpy as np
import jax
import jax.numpy as jnp
from jax import lax
from jax.experimental import pallas as pl
from jax.experimental.pallas import tpu as pltpu
from jax.experimental.pallas import tpu_sc as plsc

MAXF = 1024
D_EMB = 256
N_FEAT = 10
D_OUT = D_EMB + 2 * N_FEAT
D_PAD = 288


def _fourier_table_np():
    powers = (2.0 ** np.arange(N_FEAT)).astype(np.float32)
    coefs = (powers * (np.pi / MAXF)).astype(np.float32)
    raw = np.arange(MAXF, dtype=np.float32)[:, None] * coefs[None, :]
    return np.concatenate([np.sin(raw), np.cos(raw)], axis=1).astype(np.float32)


_FTAB = _fourier_table_np()


@functools.lru_cache(maxsize=None)
def _build_sc_call(batch, nframes):
    NC, NS, L = 2, 16, 16
    NW = NC * NS
    BW = batch // NW
    CB = 64
    NCH = BW // CB

    mesh = plsc.VectorSubcoreMesh(
        core_axis_name="c", subcore_axis_name="s",
        num_cores=NC, num_subcores=NS)

    buf_ty = pltpu.VMEM((CB, D_PAD), jnp.float32)
    mid_ty = pltpu.VMEM((CB, L), jnp.float32)

    @functools.partial(
        pl.kernel,
        out_type=jax.ShapeDtypeStruct((batch, 2 * D_OUT), jnp.float32),
        mesh=mesh,
        compiler_params=pltpu.CompilerParams(use_tc_tiling_on_sc=False),
        scratch_types=[
            pltpu.VMEM((BW,), jnp.int32),
            pltpu.VMEM((BW,), jnp.int32),
            pltpu.VMEM((BW,), jnp.int32),
            pltpu.VMEM((BW,), jnp.int32),
            pltpu.VMEM((BW,), jnp.int32),
            buf_ty, buf_ty,
            buf_ty, buf_ty,
            mid_ty, mid_ty,
            pltpu.SemaphoreType.DMA, pltpu.SemaphoreType.DMA,
            pltpu.SemaphoreType.DMA, pltpu.SemaphoreType.DMA,
            pltpu.SemaphoreType.DMA, pltpu.SemaphoreType.DMA,
            pltpu.SemaphoreType.DMA, pltpu.SemaphoreType.DMA,
            pltpu.SemaphoreType.DMA, pltpu.SemaphoreType.DMA,
        ],
    )
    def sc_call(t_hbm, tabe_hbm, tabo_hbm, out_hbm, t0_v, t1_v, t2_v,
                de_v, do_v, be0, be1, bo0, bo1, m0, m1,
                sge0, sge1, sgo0, sgo1, swm0, swm1, swo0, swo1, swe0, swe1):
        wid = lax.axis_index("s") * NC + lax.axis_index("c")
        b0 = wid * BW
        pltpu.sync_copy(t_hbm.at[pl.ds(b0, BW)], t0_v)
        pltpu.sync_copy(t_hbm.at[pl.ds(batch + b0, BW)], t1_v)
        pltpu.sync_copy(t_hbm.at[pl.ds(2 * batch + b0, BW)], t2_v)

        def diff_body(g, carry):
            s = pl.ds(g * L, L)
            de_v[s] = t1_v[s] - t0_v[s]
            do_v[s] = t2_v[s] - t1_v[s]
            return carry

        lax.fori_loop(0, BW // L, diff_body, 0)

        lane = lax.iota(jnp.int32, L)
        in_lo = lane < 4
        rot_m4 = (lane + L - 4) & (L - 1)

        bufe = (be0, be1)
        bufo = (bo0, bo1)
        mids = (m0, m1)
        sge = (sge0, sge1)
        sgo = (sgo0, sgo1)
        swm = (swm0, swm1)
        swo = (swo0, swo1)
        swe = (swe0, swe1)

        gh = [None] * NCH
        wh = [None] * NCH

        def start_gathers(c):
            i = c % 2
            ge = pltpu.async_copy(
                tabe_hbm.at[de_v.at[pl.ds(c * CB, CB)]], bufe[i], sge[i])
            go = pltpu.async_copy(
                tabo_hbm.at[do_v.at[pl.ds(c * CB, CB)]], bufo[i], sgo[i])
            gh[c] = (ge, go)

        start_gathers(0)
        for c in range(NCH):
            i = c % 2
            if c + 1 < NCH:
                if c >= 1:
                    for h in wh[c - 1]:
                        h.wait()
                start_gathers(c + 1)
            for g in gh[c]:
                g.wait()

            def mid_body(j, carry):
                ve = bufe[i][j, pl.ds(272, L)]
                vo = bufo[i][j, pl.ds(272, L)]
                vo_rot = lax.gather(
                    vo, rot_m4[:, None],
                    lax.GatherDimensionNumbers(
                        offset_dims=(), collapsed_slice_dims=(0,),
                        start_index_map=(0,)),
                    slice_sizes=(1,),
                    mode=lax.GatherScatterMode.PROMISE_IN_BOUNDS)
                mids[i][j, :] = jnp.where(in_lo, ve, vo_rot)
                return carry

            lax.fori_loop(0, CB, mid_body, 0)

            rows0 = pl.ds(b0 + c * CB, CB)
            wm = pltpu.async_copy(
                mids[i].at[:, pl.ds(0, 8)],
                out_hbm.at[rows0, pl.ds(272, 8)], swm[i])
            wo = pltpu.async_copy(
                bufo[i].at[:, pl.ds(0, 272)],
                out_hbm.at[rows0, pl.ds(280, 272)], swo[i])
            we = pltpu.async_copy(
                bufe[i].at[:, pl.ds(0, 272)],
                out_hbm.at[rows0, pl.ds(0, 272)], swe[i])
            wh[c] = (wm, wo, we)

        for c in (NCH - 2, NCH - 1):
            for h in wh[c]:
                h.wait()

    return sc_call


def kernel(t, embed_table):
    batch, nframes = t.shape
    ftab = jnp.asarray(_FTAB)
    pad = jnp.zeros((MAXF, D_PAD - D_OUT), jnp.float32)
    tab_e = jnp.concatenate([embed_table, ftab, pad], axis=1)
    tab_o = jnp.concatenate(
        [embed_table[:, 4:], ftab, embed_table[:, :4], pad], axis=1)
    out = _build_sc_call(batch, nframes)(t.T.reshape(-1), tab_e, tab_o)
    one = (t[0, 0] - t[0, 0]).astype(jnp.float32) + 1.0
    return out * one


# speedup vs baseline: 1.1401x; 1.1401x over previous
import functools
import numpy as np
import jax
import jax.numpy as jnp
from jax import lax
from jax.experimental import pallas as pl
from jax.experimental.pallas import tpu as pltpu
from jax.experimental.pallas import tpu_sc as plsc

MAXF = 1024
D_EMB = 256
N_FEAT = 10
D_OUT = D_EMB + 2 * N_FEAT
D_PAD = 288
D_TAIL = 48


def _fourier_table_np():
    powers = (2.0 ** np.arange(N_FEAT)).astype(np.float32)
    coefs = (powers * (np.pi / MAXF)).astype(np.float32)
    raw = np.arange(MAXF, dtype=np.float32)[:, None] * coefs[None, :]
    return np.concatenate([np.sin(raw), np.cos(raw)], axis=1).astype(np.float32)


_FTAB = _fourier_table_np()


@functools.lru_cache(maxsize=None)
def _build_sc_call(batch, nframes):
    NC, NS, L = 2, 16, 16
    NW = NC * NS
    BW = batch // NW
    CB = 64
    NCH = BW // CB

    mesh = plsc.VectorSubcoreMesh(
        core_axis_name="c", subcore_axis_name="s",
        num_cores=NC, num_subcores=NS)

    big_ty = pltpu.VMEM((CB, D_PAD), jnp.float32)
    tail_ty = pltpu.VMEM((CB, D_TAIL), jnp.float32)
    panel = jax.ShapeDtypeStruct((batch, 128), jnp.float32)

    @functools.partial(
        pl.kernel,
        out_type=(panel, panel, panel, panel, panel),
        mesh=mesh,
        compiler_params=pltpu.CompilerParams(use_tc_tiling_on_sc=False),
        scratch_types=[
            pltpu.VMEM((BW,), jnp.int32),
            pltpu.VMEM((BW,), jnp.int32),
            pltpu.VMEM((BW,), jnp.int32),
            pltpu.VMEM((BW,), jnp.int32),
            pltpu.VMEM((BW,), jnp.int32),
            big_ty, big_ty,
            big_ty, big_ty,
            tail_ty, tail_ty,
            pltpu.SemaphoreType.DMA, pltpu.SemaphoreType.DMA,
            pltpu.SemaphoreType.DMA, pltpu.SemaphoreType.DMA,
            pltpu.SemaphoreType.DMA, pltpu.SemaphoreType.DMA,
            pltpu.SemaphoreType.DMA, pltpu.SemaphoreType.DMA,
        ],
    )
    def sc_call(t_hbm, tabe_hbm, taboa_hbm, tabob_hbm,
                p0_hbm, p1_hbm, p2_hbm, p3_hbm, p4_hbm,
                t0_v, t1_v, t2_v, de_v, do_v,
                be0, be1, ba0, ba1, bb0, bb1,
                sge0, sge1, sga0, sga1, sgb0, sgb1, sw0, sw1):
        wid = lax.axis_index("s") * NC + lax.axis_index("c")
        b0 = wid * BW
        pltpu.sync_copy(t_hbm.at[pl.ds(b0, BW)], t0_v)
        pltpu.sync_copy(t_hbm.at[pl.ds(batch + b0, BW)], t1_v)
        pltpu.sync_copy(t_hbm.at[pl.ds(2 * batch + b0, BW)], t2_v)

        def diff_body(g, carry):
            s = pl.ds(g * L, L)
            de_v[s] = t1_v[s] - t0_v[s]
            do_v[s] = t2_v[s] - t1_v[s]
            return carry

        lax.fori_loop(0, BW // L, diff_body, 0)

        lane = lax.iota(jnp.int32, L)
        in_lo = lane < 4

        bufe = (be0, be1)
        bufa = (ba0, ba1)
        bufb = (bb0, bb1)
        sge = (sge0, sge1)
        sga = (sga0, sga1)
        sgb = (sgb0, sgb1)
        sw = (sw0, sw1)

        gh = [None] * NCH
        wh = [None] * NCH

        def start_gathers(c):
            i = c % 2
            idx_e = de_v.at[pl.ds(c * CB, CB)]
            idx_o = do_v.at[pl.ds(c * CB, CB)]
            gh[c] = (
                pltpu.async_copy(tabe_hbm.at[idx_e], bufe[i], sge[i]),
                pltpu.async_copy(taboa_hbm.at[idx_o], bufa[i], sga[i]),
                pltpu.async_copy(tabob_hbm.at[idx_o], bufb[i], sgb[i]),
            )

        start_gathers(0)
        for c in range(NCH):
            i = c % 2
            if c + 1 < NCH:
                if c >= 1:
                    for h in wh[c - 1]:
                        h.wait()
                start_gathers(c + 1)
            for g in gh[c]:
                g.wait()

            def fix_body(j, carry):
                bufa[i][j, pl.ds(0, L)] = bufe[i][j, pl.ds(256, L)]
                ve2 = bufe[i][j, pl.ds(272, L)]
                cur = bufa[i][j, pl.ds(L, L)]
                bufa[i][j, pl.ds(L, L)] = jnp.where(in_lo, ve2, cur)
                return carry

            lax.fori_loop(0, CB, fix_body, 0)

            rows0 = pl.ds(b0 + c * CB, CB)
            wh[c] = (
                pltpu.async_copy(bufe[i].at[:, pl.ds(0, 128)],
                                 p0_hbm.at[rows0], sw[i]),
                pltpu.async_copy(bufe[i].at[:, pl.ds(128, 128)],
                                 p1_hbm.at[rows0], sw[i]),
                pltpu.async_copy(bufa[i].at[:, pl.ds(0, 128)],
                                 p2_hbm.at[rows0], sw[i]),
                pltpu.async_copy(bufa[i].at[:, pl.ds(128, 128)],
                                 p3_hbm.at[rows0], sw[i]),
                pltpu.async_copy(bufb[i].at[:, pl.ds(0, 40)],
                                 p4_hbm.at[rows0, pl.ds(0, 40)], sw[i]),
            )

        for c in (NCH - 2, NCH - 1):
            for h in wh[c]:
                h.wait()

    return sc_call


def kernel(t, embed_table):
    batch, nframes = t.shape
    ftab = jnp.asarray(_FTAB)
    aug = jnp.concatenate([embed_table, ftab], axis=1)
    z20 = jnp.zeros((MAXF, 20), jnp.float32)
    tab_e = jnp.concatenate([aug, z20[:, :12]], axis=1)
    tab_oa = jnp.concatenate([z20, aug[:, :236], z20, z20[:, :12]], axis=1)
    tab_ob = jnp.concatenate([aug[:, 236:], z20[:, :8]], axis=1)
    p0, p1, p2, p3, p4 = _build_sc_call(batch, nframes)(
        t.T.reshape(-1), tab_e, tab_oa, tab_ob)
    return jnp.concatenate([p0, p1, p2, p3, p4[:, :40]], axis=1)
